# baseline (device time: 36107 ns/iter reference)
import jax
import jax.numpy as jnp
from jax import lax
from jax.experimental import pallas as pl
from jax.experimental.pallas import tpu as pltpu

N_DEV = 4
B = 2
SQ = 256
SKV_L = 256
HL = 4
DH = 64
DM = 512
BLK = 64
SQQ = SQ // N_DEV
NEG = -1e9
Q8_SCALE = 4.0 / 127


def kernel(x, Wq, K_ext, V_ext, Wo):
    def q8(t):
        return jnp.clip(jnp.round(t / Q8_SCALE), -127, 127).astype(jnp.int8)

    kg = q8(K_ext).reshape(B, SKV_L, N_DEV, HL, DH).transpose(2, 0, 3, 1, 4)
    vg = q8(V_ext).reshape(B, SKV_L, N_DEV, HL, DH).transpose(2, 0, 3, 1, 4)
    kv = jnp.stack([kg, vg], axis=1)
    xb = x.astype(jnp.bfloat16)
    wq = Wq.reshape(DM, HL, DH).transpose(1, 0, 2).astype(jnp.bfloat16)
    wo = Wo.reshape(HL, DH, DM).astype(jnp.bfloat16)

    def body(x_ref, wq_ref, kv_ref, wo_ref, out_ref,
             kv_recv, partial_q, rs_recv, ag_recv,
             kv_send_sems, kv_recv_sems,
             rs_send_sems, rs_recv_sems,
             ag_send_sems, ag_recv_sems):
        my = lax.axis_index("i")

        barrier = pltpu.get_barrier_semaphore()
        for off in (1, 2, 3):
            peer = lax.rem(my + off, N_DEV)
            pl.semaphore_signal(barrier, inc=1, device_id=(peer,),
                                device_id_type=pl.DeviceIdType.MESH)
        pl.semaphore_wait(barrier, N_DEV - 1)

        def kv_send_to(off):
            peer = lax.rem(my + off, N_DEV)
            return pltpu.make_async_remote_copy(
                src_ref=kv_ref.at[peer],
                dst_ref=kv_recv.at[my],
                send_sem=kv_send_sems.at[off - 1],
                recv_sem=kv_recv_sems.at[my],
                device_id=(peer,),
                device_id_type=pl.DeviceIdType.MESH,
            )

        kv_sends = []
        for off in (1, 3):
            d = kv_send_to(off)
            d.start()
            kv_sends.append(d)
        kv_recv[pl.ds(my, 1)] = kv_ref[pl.ds(my, 1)]

        q = []
        for b in range(B):
            qh = []
            for h in range(HL):
                qf = jnp.dot(x_ref[b], wq_ref[h],
                             preferred_element_type=jnp.float32)
                qh.append((qf * (0.125 * Q8_SCALE)).astype(jnp.bfloat16))
            q.append(qh)

        for d in kv_sends:
            d.wait_send()
        kv_diag = kv_send_to(2)
        kv_diag.start()

        qblk = lax.broadcasted_iota(jnp.int32, (SQ, SKV_L), 0) // BLK
        cblk = lax.broadcasted_iota(jnp.int32, (SQ, SKV_L), 1) // BLK

        def bias_for(c):
            kb = cblk + c * (SKV_L // BLK)
            keep = (qblk == kb) | (kb == 0) | ((qblk + kb) % 3 == 0)
            return jnp.where(keep, 0.0, NEG).astype(jnp.float32)

        m_st = [[None] * HL for _ in range(B)]
        l_st = [[None] * HL for _ in range(B)]
        ctx_st = [[None] * HL for _ in range(B)]

        def attn_chunk(c, bias, b, h, first):
            kc = kv_recv[pl.ds(c, 1), 0, b, h].reshape(
                SKV_L, DH).astype(jnp.bfloat16)
            vc = kv_recv[pl.ds(c, 1), 1, b, h].reshape(
                SKV_L, DH).astype(jnp.bfloat16)
            s = lax.dot_general(
                q[b][h], kc, (((1,), (1,)), ((), ())),
                preferred_element_type=jnp.float32) + bias
            mc = s.max(axis=1, keepdims=True)
            if first:
                m_new = mc
                p = jnp.exp(s - m_new)
                l_st[b][h] = p.sum(axis=1, keepdims=True)
                ctx_st[b][h] = jnp.dot(
                    p.astype(jnp.bfloat16), vc,
                    preferred_element_type=jnp.float32)
            else:
                m_new = jnp.maximum(m_st[b][h], mc)
                alpha = jnp.exp(m_st[b][h] - m_new)
                p = jnp.exp(s - m_new)
                l_st[b][h] = l_st[b][h] * alpha + p.sum(
                    axis=1, keepdims=True)
                ctx_st[b][h] = ctx_st[b][h] * alpha + jnp.dot(
                    p.astype(jnp.bfloat16), vc,
                    preferred_element_type=jnp.float32)
            m_st[b][h] = m_new

        def wait_chunk(c, off):
            pltpu.make_async_remote_copy(
                src_ref=kv_ref.at[c],
                dst_ref=kv_recv.at[c],
                send_sem=kv_send_sems.at[off - 1],
                recv_sem=kv_recv_sems.at[c],
                device_id=(c,),
                device_id_type=pl.DeviceIdType.MESH,
            ).wait_recv()

        for step, off in enumerate((0, 1, 3)):
            c = lax.rem(my + off, N_DEV)
            if step > 0:
                wait_chunk(c, off)
            bias = bias_for(c)
            for b in range(B):
                for h in range(HL):
                    attn_chunk(c, bias, b, h, first=(step == 0))

        def rs_send_to(off, b):
            peer = lax.rem(my + off, N_DEV)
            return pltpu.make_async_remote_copy(
                src_ref=partial_q.at[peer, b],
                dst_ref=rs_recv.at[my, b],
                send_sem=rs_send_sems.at[off - 1, b],
                recv_sem=rs_recv_sems.at[my, b],
                device_id=(peer,),
                device_id_type=pl.DeviceIdType.MESH,
            )

        c_last = lax.rem(my + 2, N_DEV)
        wait_chunk(c_last, 2)
        bias_last = bias_for(c_last)
        rs_sends = []
        for b in range(B):
            for h in range(HL):
                attn_chunk(c_last, bias_last, b, h, first=False)
            partial = jnp.zeros((SQ, DM), jnp.float32)
            for h in range(HL):
                ctx = ctx_st[b][h] * (Q8_SCALE / l_st[b][h])
                partial = partial + jnp.dot(
                    ctx.astype(jnp.bfloat16), wo_ref[h],
                    preferred_element_type=jnp.float32)
            pb = partial.astype(jnp.bfloat16)
            for qt in range(N_DEV):
                partial_q[qt, b] = pb[qt * SQQ:(qt + 1) * SQQ, :]
            for off in (1, 3, 2):
                d = rs_send_to(off, b)
                d.start()
                rs_sends.append(d)

        kv_diag.wait_send()

        red = partial_q[pl.ds(my, 1)].reshape(B, SQQ, DM).astype(jnp.float32)
        for off in (1, 3, 2):
            src = lax.rem(my + off, N_DEV)
            for b in range(B):
                pltpu.make_async_remote_copy(
                    src_ref=partial_q.at[src, b],
                    dst_ref=rs_recv.at[src, b],
                    send_sem=rs_send_sems.at[off - 1, b],
                    recv_sem=rs_recv_sems.at[src, b],
                    device_id=(src,),
                    device_id_type=pl.DeviceIdType.MESH,
                ).wait_recv()
            red = red + rs_recv[pl.ds(src, 1)].reshape(B, SQQ, DM).astype(
                jnp.float32)
        ag_recv[pl.ds(my, 1)] = red.astype(jnp.bfloat16).reshape(1, B, SQQ, DM)

        def ag_send_to(off):
            peer = lax.rem(my + off, N_DEV)
            return pltpu.make_async_remote_copy(
                src_ref=ag_recv.at[my],
                dst_ref=ag_recv.at[my],
                send_sem=ag_send_sems.at[off - 1],
                recv_sem=ag_recv_sems.at[my],
                device_id=(peer,),
                device_id_type=pl.DeviceIdType.MESH,
            )

        ag_sends = []
        for off in (1, 3, 2):
            d = ag_send_to(off)
            d.start()
            ag_sends.append(d)

        out_ref[:, pl.ds(my * SQQ, SQQ), :] = red

        for off in (1, 3, 2):
            src = lax.rem(my + off, N_DEV)
            pltpu.make_async_remote_copy(
                src_ref=ag_recv.at[src],
                dst_ref=ag_recv.at[src],
                send_sem=ag_send_sems.at[off - 1],
                recv_sem=ag_recv_sems.at[src],
                device_id=(src,),
                device_id_type=pl.DeviceIdType.MESH,
            ).wait_recv()
            out_ref[:, pl.ds(src * SQQ, SQQ), :] = ag_recv[
                pl.ds(src, 1)].reshape(B, SQQ, DM).astype(jnp.float32)

        for d in rs_sends:
            d.wait_send()
        for d in ag_sends:
            d.wait_send()

    return pl.pallas_call(
        body,
        out_shape=jax.ShapeDtypeStruct((B, SQ, DM), jnp.float32),
        in_specs=[pl.BlockSpec(memory_space=pltpu.VMEM)] * 4,
        out_specs=pl.BlockSpec(memory_space=pltpu.VMEM),
        scratch_shapes=[
            pltpu.VMEM((N_DEV, 2, B, HL, SKV_L, DH), jnp.int8),
            pltpu.VMEM((N_DEV, B, SQQ, DM), jnp.bfloat16),
            pltpu.VMEM((N_DEV, B, SQQ, DM), jnp.bfloat16),
            pltpu.VMEM((N_DEV, B, SQQ, DM), jnp.bfloat16),
            pltpu.SemaphoreType.DMA((3,)),
            pltpu.SemaphoreType.DMA((N_DEV,)),
            pltpu.SemaphoreType.DMA((3, B)),
            pltpu.SemaphoreType.DMA((N_DEV, B)),
            pltpu.SemaphoreType.DMA((3,)),
            pltpu.SemaphoreType.DMA((N_DEV,)),
        ],
        compiler_params=pltpu.CompilerParams(collective_id=0),
    )(xb, wq, kv, wo)


# device time: 34935 ns/iter; 1.0335x vs baseline; 1.0335x over previous
import jax
import jax.numpy as jnp
from jax import lax
from jax.experimental import pallas as pl
from jax.experimental.pallas import tpu as pltpu

N_DEV = 4
B = 2
SQ = 256
SKV_L = 256
HL = 4
DH = 64
DM = 512
BLK = 64
SQQ = SQ // N_DEV
NEG = -1e9
Q8_SCALE = 4.0 / 127


def kernel(x, Wq, K_ext, V_ext, Wo):
    kg = K_ext.reshape(B, SKV_L, N_DEV, HL, DH).transpose(2, 0, 3, 1, 4)
    vg = V_ext.reshape(B, SKV_L, N_DEV, HL, DH).transpose(2, 0, 3, 1, 4)
    kv = jnp.stack([kg, vg], axis=1)
    kv = jnp.clip(jnp.round(kv / Q8_SCALE), -127, 127).astype(jnp.int8)
    xb = x.astype(jnp.bfloat16)
    wq = Wq.reshape(DM, HL, DH).transpose(1, 0, 2).astype(jnp.bfloat16)
    wo = Wo.reshape(HL, DH, DM).astype(jnp.bfloat16)

    def body(x_ref, wq_ref, kv_ref, wo_ref, out_ref,
             kv_recv, partial_q, rs_recv, ag_recv,
             kv_send_sems, kv_recv_sems,
             rs_send_sems, rs_recv_sems,
             ag_send_sems, ag_recv_sems):
        my = lax.axis_index("i")

        barrier = pltpu.get_barrier_semaphore()
        for off in (1, 2, 3):
            peer = lax.rem(my + off, N_DEV)
            pl.semaphore_signal(barrier, inc=1, device_id=(peer,),
                                device_id_type=pl.DeviceIdType.MESH)
        pl.semaphore_wait(barrier, N_DEV - 1)

        def kv_send_to(off):
            peer = lax.rem(my + off, N_DEV)
            return pltpu.make_async_remote_copy(
                src_ref=kv_ref.at[peer],
                dst_ref=kv_recv.at[my],
                send_sem=kv_send_sems.at[off - 1],
                recv_sem=kv_recv_sems.at[my],
                device_id=(peer,),
                device_id_type=pl.DeviceIdType.MESH,
            )

        kv_sends = []
        for off in (1, 3):
            d = kv_send_to(off)
            d.start()
            kv_sends.append(d)
        kv_recv[pl.ds(my, 1)] = kv_ref[pl.ds(my, 1)]

        q = []
        for b in range(B):
            qh = []
            for h in range(HL):
                qf = jnp.dot(x_ref[b], wq_ref[h],
                             preferred_element_type=jnp.float32)
                qh.append((qf * (0.125 * Q8_SCALE)).astype(jnp.bfloat16))
            q.append(qh)

        for d in kv_sends:
            d.wait_send()
        kv_diag = kv_send_to(2)
        kv_diag.start()

        qblk = lax.broadcasted_iota(jnp.int32, (SQ, SKV_L), 0) // BLK
        cblk = lax.broadcasted_iota(jnp.int32, (SQ, SKV_L), 1) // BLK

        def bias_for(c):
            kb = cblk + c * (SKV_L // BLK)
            keep = (qblk == kb) | (kb == 0) | ((qblk + kb) % 3 == 0)
            return jnp.where(keep, 0.0, NEG).astype(jnp.float32)

        m_st = [[None] * HL for _ in range(B)]
        l_st = [[None] * HL for _ in range(B)]
        ctx_st = [[None] * HL for _ in range(B)]

        def attn_chunk(c, bias, b, h, first):
            kc = kv_recv[pl.ds(c, 1), 0, b, h].reshape(
                SKV_L, DH).astype(jnp.bfloat16)
            vc = kv_recv[pl.ds(c, 1), 1, b, h].reshape(
                SKV_L, DH).astype(jnp.bfloat16)
            s = lax.dot_general(
                q[b][h], kc, (((1,), (1,)), ((), ())),
                preferred_element_type=jnp.float32) + bias
            mc = s.max(axis=1, keepdims=True)
            if first:
                m_new = mc
                p = jnp.exp(s - m_new)
                l_st[b][h] = p.sum(axis=1, keepdims=True)
                ctx_st[b][h] = jnp.dot(
                    p.astype(jnp.bfloat16), vc,
                    preferred_element_type=jnp.float32)
            else:
                m_new = jnp.maximum(m_st[b][h], mc)
                alpha = jnp.exp(m_st[b][h] - m_new)
                p = jnp.exp(s - m_new)
                l_st[b][h] = l_st[b][h] * alpha + p.sum(
                    axis=1, keepdims=True)
                ctx_st[b][h] = ctx_st[b][h] * alpha + jnp.dot(
                    p.astype(jnp.bfloat16), vc,
                    preferred_element_type=jnp.float32)
            m_st[b][h] = m_new

        def wait_chunk(c, off):
            pltpu.make_async_remote_copy(
                src_ref=kv_ref.at[c],
                dst_ref=kv_recv.at[c],
                send_sem=kv_send_sems.at[off - 1],
                recv_sem=kv_recv_sems.at[c],
                device_id=(c,),
                device_id_type=pl.DeviceIdType.MESH,
            ).wait_recv()

        for step, off in enumerate((0, 1, 3)):
            c = lax.rem(my + off, N_DEV)
            if step > 0:
                wait_chunk(c, off)
            bias = bias_for(c)
            for b in range(B):
                for h in range(HL):
                    attn_chunk(c, bias, b, h, first=(step == 0))

        def rs_send_to(off, b):
            peer = lax.rem(my + off, N_DEV)
            return pltpu.make_async_remote_copy(
                src_ref=partial_q.at[peer, b],
                dst_ref=rs_recv.at[my, b],
                send_sem=rs_send_sems.at[off - 1, b],
                recv_sem=rs_recv_sems.at[my, b],
                device_id=(peer,),
                device_id_type=pl.DeviceIdType.MESH,
            )

        c_last = lax.rem(my + 2, N_DEV)
        wait_chunk(c_last, 2)
        bias_last = bias_for(c_last)
        rs_sends = []
        for b in range(B):
            for h in range(HL):
                attn_chunk(c_last, bias_last, b, h, first=False)
            partial = jnp.zeros((SQ, DM), jnp.float32)
            for h in range(HL):
                ctx = ctx_st[b][h] * (Q8_SCALE / l_st[b][h])
                partial = partial + jnp.dot(
                    ctx.astype(jnp.bfloat16), wo_ref[h],
                    preferred_element_type=jnp.float32)
            pb = partial.astype(jnp.bfloat16)
            for qt in range(N_DEV):
                partial_q[qt, b] = pb[qt * SQQ:(qt + 1) * SQQ, :]
            for off in (1, 3, 2):
                d = rs_send_to(off, b)
                d.start()
                rs_sends.append(d)

        kv_diag.wait_send()

        red = partial_q[pl.ds(my, 1)].reshape(B, SQQ, DM).astype(jnp.float32)
        for off in (1, 3, 2):
            src = lax.rem(my + off, N_DEV)
            for b in range(B):
                pltpu.make_async_remote_copy(
                    src_ref=partial_q.at[src, b],
                    dst_ref=rs_recv.at[src, b],
                    send_sem=rs_send_sems.at[off - 1, b],
                    recv_sem=rs_recv_sems.at[src, b],
                    device_id=(src,),
                    device_id_type=pl.DeviceIdType.MESH,
                ).wait_recv()
            red = red + rs_recv[pl.ds(src, 1)].reshape(B, SQQ, DM).astype(
                jnp.float32)
        ag_recv[pl.ds(my, 1)] = red.astype(jnp.bfloat16).reshape(1, B, SQQ, DM)

        def ag_send_to(off):
            peer = lax.rem(my + off, N_DEV)
            return pltpu.make_async_remote_copy(
                src_ref=ag_recv.at[my],
                dst_ref=ag_recv.at[my],
                send_sem=ag_send_sems.at[off - 1],
                recv_sem=ag_recv_sems.at[my],
                device_id=(peer,),
                device_id_type=pl.DeviceIdType.MESH,
            )

        ag_sends = []
        for off in (1, 3, 2):
            d = ag_send_to(off)
            d.start()
            ag_sends.append(d)

        out_ref[:, pl.ds(my * SQQ, SQQ), :] = red

        for off in (1, 3, 2):
            src = lax.rem(my + off, N_DEV)
            pltpu.make_async_remote_copy(
                src_ref=ag_recv.at[src],
                dst_ref=ag_recv.at[src],
                send_sem=ag_send_sems.at[off - 1],
                recv_sem=ag_recv_sems.at[src],
                device_id=(src,),
                device_id_type=pl.DeviceIdType.MESH,
            ).wait_recv()
            out_ref[:, pl.ds(src * SQQ, SQQ), :] = ag_recv[
                pl.ds(src, 1)].reshape(B, SQQ, DM).astype(jnp.float32)

        for d in rs_sends:
            d.wait_send()
        for d in ag_sends:
            d.wait_send()

    return pl.pallas_call(
        body,
        out_shape=jax.ShapeDtypeStruct((B, SQ, DM), jnp.float32),
        in_specs=[pl.BlockSpec(memory_space=pltpu.VMEM)] * 4,
        out_specs=pl.BlockSpec(memory_space=pltpu.VMEM),
        scratch_shapes=[
            pltpu.VMEM((N_DEV, 2, B, HL, SKV_L, DH), jnp.int8),
            pltpu.VMEM((N_DEV, B, SQQ, DM), jnp.bfloat16),
            pltpu.VMEM((N_DEV, B, SQQ, DM), jnp.bfloat16),
            pltpu.VMEM((N_DEV, B, SQQ, DM), jnp.bfloat16),
            pltpu.SemaphoreType.DMA((3,)),
            pltpu.SemaphoreType.DMA((N_DEV,)),
            pltpu.SemaphoreType.DMA((3, B)),
            pltpu.SemaphoreType.DMA((N_DEV, B)),
            pltpu.SemaphoreType.DMA((3,)),
            pltpu.SemaphoreType.DMA((N_DEV,)),
        ],
        compiler_params=pltpu.CompilerParams(collective_id=0),
    )(xb, wq, kv, wo)
